# baseline (speedup 1.0000x reference)
import functools

import jax
import jax.numpy as jnp
from jax import lax
from jax.experimental import pallas as pl
from jax.experimental.pallas import tpu as pltpu

N = 10000
BN = 1000
NB = N // BN
E = 160000

_INTERPRET = False


def _lrelu(x, s):
    return jnp.where(x >= 0, x, s * x)


def _elu(x):
    return jnp.where(x > 0, x, jnp.exp(jnp.minimum(x, 0.0)) - 1.0)


_AE_ACT = (1, 1, 1, 0, 0, 1, 1, 1, 1, 0)


def _ae_body(*refs):
    x_ref = refs[0]
    out_ref = refs[21]
    h = x_ref[...]
    for i in range(10):
        W = refs[1 + 2 * i][...]
        b = refs[2 + 2 * i][...]
        h = jnp.dot(h, W, preferred_element_type=jnp.float32) + b
        if _AE_ACT[i]:
            h = jnp.maximum(h, 0.0)
    out_ref[...] = h


def _ae(x, Ws, bs):
    ins = [x]
    specs = [pl.BlockSpec((BN, 128), lambda i: (i, 0))]
    for W, b in zip(Ws, bs):
        ins.append(W)
        ins.append(b.reshape(1, -1))
        specs.append(pl.BlockSpec(W.shape, lambda i: (0, 0)))
        specs.append(pl.BlockSpec((1, b.size), lambda i: (0, 0)))
    return pl.pallas_call(
        _ae_body,
        grid=(NB,),
        in_specs=specs,
        out_specs=pl.BlockSpec((BN, 128), lambda i: (i, 0)),
        out_shape=jax.ShapeDtypeStruct((N, 128), jnp.float32),
        interpret=_INTERPRET,
    )(*ins)


def _stats_body(za_ref, zb_ref, out_ref):
    t = _elu(_lrelu(jnp.concatenate([za_ref[...], zb_ref[...]], axis=1), 0.2))
    s1 = jnp.sum(t, axis=0, keepdims=True)
    s2 = jnp.sum(t * t, axis=0, keepdims=True)

    @pl.when(pl.program_id(0) == 0)
    def _():
        out_ref[...] = jnp.zeros_like(out_ref)

    out_ref[...] += jnp.concatenate([s1, s2], axis=0)


def _stats(za, ca, zb, cb, K2):
    return pl.pallas_call(
        _stats_body,
        grid=(NB,),
        in_specs=[
            pl.BlockSpec((BN, K2), lambda i, c=ca: (i, c)),
            pl.BlockSpec((BN, K2), lambda i, c=cb: (i, c)),
        ],
        out_specs=pl.BlockSpec((2, 2 * K2), lambda i: (0, 0)),
        out_shape=jax.ShapeDtypeStruct((2, 2 * K2), jnp.float32),
        interpret=_INTERPRET,
    )(za, zb)


def _dense_body(mode, Ko2, *refs):
    i = 0
    za_ref = refs[i]; i += 1
    zb_ref = refs[i]; i += 1
    sums_ref = None
    if mode in ("norm", "head"):
        sums_ref = refs[i]; i += 1
    W_ref = refs[i]; i += 1
    b_ref = None
    if mode == "head":
        b_ref = refs[i]; i += 1
    h = jnp.concatenate([za_ref[...], zb_ref[...]], axis=1)
    if mode in ("norm", "head"):
        t = _elu(_lrelu(h, 0.2))
        s = sums_ref[...]
        m = s[0:1, :] / N
        var = s[1:2, :] / N - m * m
        h = (t - m) * lax.rsqrt(var + 1e-5)
    y = jnp.dot(h, W_ref[...], preferred_element_type=jnp.float32)
    if mode == "head":
        y = _lrelu(y + b_ref[...], 0.01)
        y = y - jnp.max(y, axis=1, keepdims=True)
        ey = jnp.exp(y)
        refs[i][...] = ey / jnp.sum(ey, axis=1, keepdims=True)
    else:
        refs[i][...] = y[:, :Ko2]
        refs[i + 1][...] = y[:, Ko2:]


def _dense(mode, za, ca, zb, cb, K2, sums, W, b=None):
    Kin, Kout = W.shape
    ins = [za, zb]
    specs = [
        pl.BlockSpec((BN, K2), lambda i, c=ca: (i, c)),
        pl.BlockSpec((BN, K2), lambda i, c=cb: (i, c)),
    ]
    if mode in ("norm", "head"):
        ins.append(sums)
        specs.append(pl.BlockSpec((2, Kin), lambda i: (0, 0)))
    ins.append(W)
    specs.append(pl.BlockSpec((Kin, Kout), lambda i: (0, 0)))
    if mode == "head":
        ins.append(b.reshape(1, -1))
        specs.append(pl.BlockSpec((1, Kout), lambda i: (0, 0)))
        out_specs = pl.BlockSpec((BN, Kout), lambda i: (i, 0))
        out_shape = jax.ShapeDtypeStruct((N, Kout), jnp.float32)
    else:
        Ko2 = Kout // 2
        out_specs = [
            pl.BlockSpec((BN, Ko2), lambda i: (i, 0)),
            pl.BlockSpec((BN, Ko2), lambda i: (i, 0)),
        ]
        out_shape = [
            jax.ShapeDtypeStruct((N, Ko2), jnp.float32),
            jax.ShapeDtypeStruct((N, Ko2), jnp.float32),
        ]
    return pl.pallas_call(
        functools.partial(_dense_body, mode, Kout // 2),
        grid=(NB,),
        in_specs=specs,
        out_specs=out_specs,
        out_shape=out_shape,
        interpret=_INTERPRET,
    )(*ins)


def _spmm(sa, sb, src, dst, w):
    ga = jax.ops.segment_sum(jnp.take(sa, src, axis=0) * w[:, None], dst,
                             num_segments=N)
    gb = jax.ops.segment_sum(jnp.take(sb, src, axis=0) * w[:, None], dst,
                             num_segments=N)
    return ga, gb


def kernel(x, edge_index, edge_weight,
           ae_enc1_W, ae_enc1_b, ae_enc2_W, ae_enc2_b, ae_enc3_W, ae_enc3_b,
           ae_z1_W, ae_z1_b, ae_z2_W, ae_z2_b,
           ae_dec0_W, ae_dec0_b, ae_dec1_W, ae_dec1_b, ae_dec2_W, ae_dec2_b,
           ae_dec3_W, ae_dec3_b, ae_xbar_W, ae_xbar_b,
           gcn1_W, gcn2_W, gcn3_W, gcn4_W,
           mlp_W, mlp_b):
    src = edge_index[0]
    dst = edge_index[1]
    w = edge_weight

    x_bar = _ae(
        x,
        [ae_enc1_W, ae_enc2_W, ae_enc3_W, ae_z1_W, ae_z2_W,
         ae_dec0_W, ae_dec1_W, ae_dec2_W, ae_dec3_W, ae_xbar_W],
        [ae_enc1_b, ae_enc2_b, ae_enc3_b, ae_z1_b, ae_z2_b,
         ae_dec0_b, ae_dec1_b, ae_dec2_b, ae_dec3_b, ae_xbar_b],
    )

    xa = x[:, :64] + 0.0
    xb = x[:, 64:] + 0.0
    g1a, g1b = _spmm(xa, xb, src, dst, w)
    z1a, z1b = _dense("plain", g1a, 0, g1b, 0, 64, None, gcn1_W)
    sums1 = _stats(z1a, 0, z1b, 0, 256)

    s2a, s2b = _dense("norm", z1a, 0, z1b, 0, 256, sums1, gcn2_W)
    g2a, g2b = _spmm(s2a, s2b, src, dst, w)
    sums2 = _stats(g2a, 0, g2b, 0, 128)

    s3a, s3b = _dense("norm", g2a, 0, g2b, 0, 128, sums2, gcn3_W)
    g3a, g3b = _spmm(s3a, s3b, src, dst, w)
    sums3 = _stats(g3a, 0, g3b, 0, 128)

    s4a, s4b = _dense("norm", g3a, 0, g3b, 0, 128, sums3, gcn4_W)
    g4a, g4b = _spmm(s4a, s4b, src, dst, w)
    sums4 = _stats(g4a, 0, g4b, 0, 32)

    predict = _dense("head", g4a, 0, g4b, 0, 32, sums4, mlp_W, mlp_b)
    return jnp.concatenate([x_bar, predict], axis=1)


# speedup vs baseline: 5.8799x; 5.8799x over previous
import functools

import jax
import jax.numpy as jnp
from jax import lax
from jax.experimental import pallas as pl
from jax.experimental.pallas import tpu as pltpu
from jax.experimental.pallas import tpu_sc as plsc

N = 10000
BN = 1000
NB = N // BN
E = 160000

_INTERPRET = False


def _lrelu(x, s):
    return jnp.where(x >= 0, x, s * x)


def _elu(x):
    return jnp.where(x > 0, x, jnp.exp(jnp.minimum(x, 0.0)) - 1.0)


_AE_ACT = (1, 1, 1, 0, 0, 1, 1, 1, 1, 0)


def _ae_body(*refs):
    x_ref = refs[0]
    out_ref = refs[21]
    h = x_ref[...]
    for i in range(10):
        W = refs[1 + 2 * i][...]
        b = refs[2 + 2 * i][...]
        h = jnp.dot(h, W, preferred_element_type=jnp.float32) + b
        if _AE_ACT[i]:
            h = jnp.maximum(h, 0.0)
    out_ref[...] = h


def _ae(x, Ws, bs):
    ins = [x]
    specs = [pl.BlockSpec((BN, 128), lambda i: (i, 0))]
    for W, b in zip(Ws, bs):
        ins.append(W)
        ins.append(b.reshape(1, -1))
        specs.append(pl.BlockSpec(W.shape, lambda i: (0, 0)))
        specs.append(pl.BlockSpec((1, b.size), lambda i: (0, 0)))
    return pl.pallas_call(
        _ae_body,
        grid=(NB,),
        in_specs=specs,
        out_specs=pl.BlockSpec((BN, 128), lambda i: (i, 0)),
        out_shape=jax.ShapeDtypeStruct((N, 128), jnp.float32),
        interpret=_INTERPRET,
    )(*ins)


def _stats_body(combine, za_ref, zb_ref, out_ref):
    if combine == "cat":
        z = jnp.concatenate([za_ref[...], zb_ref[...]], axis=1)
    else:
        z = za_ref[...] + zb_ref[...]
    t = _elu(_lrelu(z, 0.2))
    s1 = jnp.sum(t, axis=0, keepdims=True)
    s2 = jnp.sum(t * t, axis=0, keepdims=True)

    @pl.when(pl.program_id(0) == 0)
    def _():
        out_ref[...] = jnp.zeros_like(out_ref)

    out_ref[...] += jnp.concatenate([s1, s2], axis=0)


def _stats(combine, za, ca, zb, cb, K2):
    K = 2 * K2 if combine == "cat" else K2
    return pl.pallas_call(
        functools.partial(_stats_body, combine),
        grid=(NB,),
        in_specs=[
            pl.BlockSpec((BN, K2), lambda i, c=ca: (i, c)),
            pl.BlockSpec((BN, K2), lambda i, c=cb: (i, c)),
        ],
        out_specs=pl.BlockSpec((2, K), lambda i: (0, 0)),
        out_shape=jax.ShapeDtypeStruct((2, K), jnp.float32),
        interpret=_INTERPRET,
    )(za, zb)


def _dense_body(mode, combine, Ko2, *refs):
    i = 0
    za_ref = refs[i]; i += 1
    zb_ref = refs[i]; i += 1
    sums_ref = None
    if mode in ("norm", "single", "head"):
        sums_ref = refs[i]; i += 1
    W_ref = refs[i]; i += 1
    b_ref = None
    if mode == "head":
        b_ref = refs[i]; i += 1
    if combine == "cat":
        h = jnp.concatenate([za_ref[...], zb_ref[...]], axis=1)
    else:
        h = za_ref[...] + zb_ref[...]
    if mode in ("norm", "single", "head"):
        t = _elu(_lrelu(h, 0.2))
        s = sums_ref[...]
        m = s[0:1, :] / N
        var = s[1:2, :] / N - m * m
        h = (t - m) * lax.rsqrt(var + 1e-5)
    y = jnp.dot(h, W_ref[...], preferred_element_type=jnp.float32)
    if mode == "head":
        y = _lrelu(y + b_ref[...], 0.01)
        y = y - jnp.max(y, axis=1, keepdims=True)
        ey = jnp.exp(y)
        refs[i][...] = ey / jnp.sum(ey, axis=1, keepdims=True)
    elif mode == "single":
        refs[i][...] = y
    else:
        refs[i][...] = y[:, :Ko2]
        refs[i + 1][...] = y[:, Ko2:]


def _dense(mode, combine, za, ca, zb, cb, K2, sums, W, b=None):
    Kin, Kout = W.shape
    ins = [za, zb]
    specs = [
        pl.BlockSpec((BN, K2), lambda i, c=ca: (i, c)),
        pl.BlockSpec((BN, K2), lambda i, c=cb: (i, c)),
    ]
    if mode in ("norm", "single", "head"):
        ins.append(sums)
        specs.append(pl.BlockSpec((2, Kin), lambda i: (0, 0)))
    ins.append(W)
    specs.append(pl.BlockSpec((Kin, Kout), lambda i: (0, 0)))
    if mode == "head":
        ins.append(b.reshape(1, -1))
        specs.append(pl.BlockSpec((1, Kout), lambda i: (0, 0)))
    if mode in ("head", "single"):
        out_specs = pl.BlockSpec((BN, Kout), lambda i: (i, 0))
        out_shape = jax.ShapeDtypeStruct((N, Kout), jnp.float32)
    else:
        Ko2 = Kout // 2
        out_specs = [
            pl.BlockSpec((BN, Ko2), lambda i: (i, 0)),
            pl.BlockSpec((BN, Ko2), lambda i: (i, 0)),
        ]
        out_shape = [
            jax.ShapeDtypeStruct((N, Ko2), jnp.float32),
            jax.ShapeDtypeStruct((N, Ko2), jnp.float32),
        ]
    return pl.pallas_call(
        functools.partial(_dense_body, mode, combine, Kout // 2),
        grid=(NB,),
        in_specs=specs,
        out_specs=out_specs,
        out_shape=out_shape,
        interpret=_INTERPRET,
    )(*ins)


_KH = 128
_CH = 125
_NSUB = 16
_RPT = 624
_NTAIL = N - _NSUB * _RPT
_ZR = 24


def _splat(vec16, j):
    idx = jnp.full((16, 1), j, dtype=jnp.int32)
    return lax.gather(
        vec16, idx,
        dimension_numbers=lax.GatherDimensionNumbers(
            offset_dims=(), collapsed_slice_dims=(0,), start_index_map=(0,)),
        slice_sizes=(1,), mode=lax.GatherScatterMode.PROMISE_IN_BOUNDS)


def _spmm_body(mode, nchunk, sa_hbm, sb_hbm, src_hbm, dst_hbm, w_hbm,
               oa_hbm, ob_hbm, src_v, dst_v, w_v, rows_v, zb_v, acc_sh, sem):
    c = lax.axis_index("c")
    s = lax.axis_index("s")
    ept = nchunk * _CH
    row0 = pl.multiple_of(s * _RPT, 8)

    def zrow(i, carry):
        for kk in range(8):
            zb_v[i, pl.ds(kk * 16, 16)] = jnp.zeros((16,), jnp.float32)
        return carry
    lax.fori_loop(0, _ZR, zrow, 0)

    def zcopy(i, carry):
        off = pl.multiple_of(row0 + i * _ZR, 8)
        pltpu.sync_copy(zb_v, acc_sh.at[pl.ds(off, _ZR)])
        return carry
    lax.fori_loop(0, _RPT // _ZR, zcopy, 0)

    @pl.when(s == 0)
    def _():
        pltpu.sync_copy(zb_v.at[pl.ds(0, _NTAIL)],
                        acc_sh.at[pl.ds(_NSUB * _RPT, _NTAIL)])

    wid = c * _NSUB + s if mode == "edge" else s
    pltpu.sync_copy(src_hbm.at[wid], src_v)
    pltpu.sync_copy(dst_hbm.at[wid], dst_v)
    woff = pl.multiple_of(wid * ept, 8)
    pltpu.sync_copy(w_hbm.at[pl.ds(woff, ept)], w_v.at[pl.ds(0, ept)])
    plsc.subcore_barrier()

    def chunk(ci, carry):
        if mode == "edge":
            pltpu.async_copy(sa_hbm.at[src_v.at[ci]], rows_v, sem).wait()
        else:
            @pl.when(c == 0)
            def _():
                pltpu.async_copy(sa_hbm.at[src_v.at[ci]], rows_v, sem).wait()

            @pl.when(c == 1)
            def _():
                pltpu.async_copy(sb_hbm.at[src_v.at[ci]], rows_v, sem).wait()

        def scale_e(e, carry2):
            j = jnp.bitwise_and(e, 15)
            w16 = w_v[pl.ds(ci * _CH + e - j, 16)]
            wv = _splat(w16, j)
            for kk in range(8):
                rows_v[e, pl.ds(kk * 16, 16)] = (
                    rows_v[e, pl.ds(kk * 16, 16)] * wv)
            return carry2
        lax.fori_loop(0, _CH, scale_e, 0)

        pltpu.sync_copy(rows_v, acc_sh.at[dst_v.at[ci]], add=True)
        return carry
    lax.fori_loop(0, nchunk, chunk, 0)
    plsc.subcore_barrier()

    @pl.when(c == 0)
    def _():
        pltpu.sync_copy(acc_sh.at[pl.ds(row0, _RPT)],
                        oa_hbm.at[pl.ds(row0, _RPT)])

        @pl.when(s == 0)
        def _():
            pltpu.sync_copy(acc_sh.at[pl.ds(_NSUB * _RPT, _NTAIL)],
                            oa_hbm.at[pl.ds(_NSUB * _RPT, _NTAIL)])

    @pl.when(c == 1)
    def _():
        pltpu.sync_copy(acc_sh.at[pl.ds(row0, _RPT)],
                        ob_hbm.at[pl.ds(row0, _RPT)])

        @pl.when(s == 0)
        def _():
            pltpu.sync_copy(acc_sh.at[pl.ds(_NSUB * _RPT, _NTAIL)],
                            ob_hbm.at[pl.ds(_NSUB * _RPT, _NTAIL)])


def _spmm(mode, sa, sb, src3, dst3, w):
    nworker = 32 if mode == "edge" else _NSUB
    nchunk = E // (nworker * _CH)
    ept = nchunk * _CH
    mesh = plsc.VectorSubcoreMesh(core_axis_name="c", subcore_axis_name="s")
    f = pl.kernel(
        functools.partial(_spmm_body, mode, nchunk),
        out_type=[jax.ShapeDtypeStruct((N, _KH), jnp.float32),
                  jax.ShapeDtypeStruct((N, _KH), jnp.float32)],
        mesh=mesh,
        scratch_types=[
            pltpu.VMEM((nchunk, _CH), jnp.int32),
            pltpu.VMEM((nchunk, _CH), jnp.int32),
            pltpu.VMEM((ept + 16,), jnp.float32),
            pltpu.VMEM((_CH, _KH), jnp.float32),
            pltpu.VMEM((_ZR, _KH), jnp.float32),
            pltpu.VMEM_SHARED((N, _KH), jnp.float32),
            pltpu.SemaphoreType.DMA,
        ],
    )
    return f(sa, sb, src3, dst3, w)


def kernel(x, edge_index, edge_weight,
           ae_enc1_W, ae_enc1_b, ae_enc2_W, ae_enc2_b, ae_enc3_W, ae_enc3_b,
           ae_z1_W, ae_z1_b, ae_z2_W, ae_z2_b,
           ae_dec0_W, ae_dec0_b, ae_dec1_W, ae_dec1_b, ae_dec2_W, ae_dec2_b,
           ae_dec3_W, ae_dec3_b, ae_xbar_W, ae_xbar_b,
           gcn1_W, gcn2_W, gcn3_W, gcn4_W,
           mlp_W, mlp_b):
    src_e = edge_index[0].reshape(_NSUB, -1, _CH)
    dst_e = edge_index[1].reshape(_NSUB, -1, _CH)
    src_w = edge_index[0].reshape(32, -1, _CH)
    dst_w = edge_index[1].reshape(32, -1, _CH)
    w = edge_weight

    x_bar = _ae(
        x,
        [ae_enc1_W, ae_enc2_W, ae_enc3_W, ae_z1_W, ae_z2_W,
         ae_dec0_W, ae_dec1_W, ae_dec2_W, ae_dec3_W, ae_xbar_W],
        [ae_enc1_b, ae_enc2_b, ae_enc3_b, ae_z1_b, ae_z2_b,
         ae_dec0_b, ae_dec1_b, ae_dec2_b, ae_dec3_b, ae_xbar_b],
    )

    g1a, g1b = _spmm("edge", x, x, src_w, dst_w, w)
    z1a, z1b = _dense("plain", "add", g1a, 0, g1b, 0, 128, None, gcn1_W)
    sums1 = _stats("cat", z1a, 0, z1b, 0, 256)

    s2a, s2b = _dense("norm", "cat", z1a, 0, z1b, 0, 256, sums1, gcn2_W)
    g2a, g2b = _spmm("feat", s2a, s2b, src_e, dst_e, w)
    sums2 = _stats("cat", g2a, 0, g2b, 0, 128)

    s3a, s3b = _dense("norm", "cat", g2a, 0, g2b, 0, 128, sums2, gcn3_W)
    g3a, g3b = _spmm("feat", s3a, s3b, src_e, dst_e, w)
    sums3 = _stats("cat", g3a, 0, g3b, 0, 128)

    W4p = jnp.pad(gcn4_W, ((0, 0), (0, _KH - gcn4_W.shape[1])))
    s4 = _dense("single", "cat", g3a, 0, g3b, 0, 128, sums3, W4p)
    g4a, g4b = _spmm("edge", s4, s4, src_w, dst_w, w)
    sums4 = _stats("add", g4a, 0, g4b, 0, 128)

    mlp_Wp = jnp.pad(mlp_W, ((0, _KH - mlp_W.shape[0]), (0, 0)))
    predict = _dense("head", "add", g4a, 0, g4b, 0, 128, sums4, mlp_Wp, mlp_b)
    return jnp.concatenate([x_bar, predict], axis=1)


# speedup vs baseline: 9.6086x; 1.6341x over previous
import functools

import jax
import jax.numpy as jnp
from jax import lax
from jax.experimental import pallas as pl
from jax.experimental.pallas import tpu as pltpu
from jax.experimental.pallas import tpu_sc as plsc

N = 10000
BN = 1000
NB = N // BN
E = 160000

_INTERPRET = False


def _lrelu(x, s):
    return jnp.where(x >= 0, x, s * x)


def _elu(x):
    return jnp.where(x > 0, x, jnp.exp(jnp.minimum(x, 0.0)) - 1.0)


_AE_ACT = (1, 1, 1, 0, 0, 1, 1, 1, 1, 0)


def _ae_body(*refs):
    x_ref = refs[0]
    out_ref = refs[21]
    h = x_ref[...]
    for i in range(10):
        W = refs[1 + 2 * i][...]
        b = refs[2 + 2 * i][...]
        h = jnp.dot(h, W, preferred_element_type=jnp.float32) + b
        if _AE_ACT[i]:
            h = jnp.maximum(h, 0.0)
    out_ref[...] = h


def _ae(x, Ws, bs):
    ins = [x]
    specs = [pl.BlockSpec((BN, 128), lambda i: (i, 0))]
    for W, b in zip(Ws, bs):
        ins.append(W)
        ins.append(b.reshape(1, -1))
        specs.append(pl.BlockSpec(W.shape, lambda i: (0, 0)))
        specs.append(pl.BlockSpec((1, b.size), lambda i: (0, 0)))
    return pl.pallas_call(
        _ae_body,
        grid=(NB,),
        in_specs=specs,
        out_specs=pl.BlockSpec((BN, 128), lambda i: (i, 0)),
        out_shape=jax.ShapeDtypeStruct((N, 128), jnp.float32),
        interpret=_INTERPRET,
    )(*ins)


def _stats_body(combine, za_ref, zb_ref, out_ref):
    if combine == "cat":
        z = jnp.concatenate([za_ref[...], zb_ref[...]], axis=1)
    else:
        z = za_ref[...] + zb_ref[...]
    t = _elu(_lrelu(z, 0.2))
    s1 = jnp.sum(t, axis=0, keepdims=True)
    s2 = jnp.sum(t * t, axis=0, keepdims=True)

    @pl.when(pl.program_id(0) == 0)
    def _():
        out_ref[...] = jnp.zeros_like(out_ref)

    out_ref[...] += jnp.concatenate([s1, s2], axis=0)


def _stats(combine, za, ca, zb, cb, K2):
    K = 2 * K2 if combine == "cat" else K2
    return pl.pallas_call(
        functools.partial(_stats_body, combine),
        grid=(NB,),
        in_specs=[
            pl.BlockSpec((BN, K2), lambda i, c=ca: (i, c)),
            pl.BlockSpec((BN, K2), lambda i, c=cb: (i, c)),
        ],
        out_specs=pl.BlockSpec((2, K), lambda i: (0, 0)),
        out_shape=jax.ShapeDtypeStruct((2, K), jnp.float32),
        interpret=_INTERPRET,
    )(za, zb)


def _dense_body(mode, combine, Ko2, *refs):
    i = 0
    za_ref = refs[i]; i += 1
    zb_ref = refs[i]; i += 1
    sums_ref = None
    if mode in ("norm", "single", "head"):
        sums_ref = refs[i]; i += 1
    W_ref = refs[i]; i += 1
    b_ref = None
    if mode == "head":
        b_ref = refs[i]; i += 1
    if combine == "cat":
        h = jnp.concatenate([za_ref[...], zb_ref[...]], axis=1)
    else:
        h = za_ref[...] + zb_ref[...]
    if mode in ("norm", "single", "head"):
        t = _elu(_lrelu(h, 0.2))
        s = sums_ref[...]
        m = s[0:1, :] / N
        var = s[1:2, :] / N - m * m
        h = (t - m) * lax.rsqrt(var + 1e-5)
    y = jnp.dot(h, W_ref[...], preferred_element_type=jnp.float32)
    if mode == "head":
        y = _lrelu(y + b_ref[...], 0.01)
        y = y - jnp.max(y, axis=1, keepdims=True)
        ey = jnp.exp(y)
        refs[i][...] = ey / jnp.sum(ey, axis=1, keepdims=True)
    elif mode == "single":
        refs[i][...] = y
    else:
        refs[i][...] = y[:, :Ko2]
        refs[i + 1][...] = y[:, Ko2:]


def _dense(mode, combine, za, ca, zb, cb, K2, sums, W, b=None):
    Kin, Kout = W.shape
    ins = [za, zb]
    specs = [
        pl.BlockSpec((BN, K2), lambda i, c=ca: (i, c)),
        pl.BlockSpec((BN, K2), lambda i, c=cb: (i, c)),
    ]
    if mode in ("norm", "single", "head"):
        ins.append(sums)
        specs.append(pl.BlockSpec((2, Kin), lambda i: (0, 0)))
    ins.append(W)
    specs.append(pl.BlockSpec((Kin, Kout), lambda i: (0, 0)))
    if mode == "head":
        ins.append(b.reshape(1, -1))
        specs.append(pl.BlockSpec((1, Kout), lambda i: (0, 0)))
    if mode in ("head", "single"):
        out_specs = pl.BlockSpec((BN, Kout), lambda i: (i, 0))
        out_shape = jax.ShapeDtypeStruct((N, Kout), jnp.float32)
    else:
        Ko2 = Kout // 2
        out_specs = [
            pl.BlockSpec((BN, Ko2), lambda i: (i, 0)),
            pl.BlockSpec((BN, Ko2), lambda i: (i, 0)),
        ]
        out_shape = [
            jax.ShapeDtypeStruct((N, Ko2), jnp.float32),
            jax.ShapeDtypeStruct((N, Ko2), jnp.float32),
        ]
    return pl.pallas_call(
        functools.partial(_dense_body, mode, combine, Kout // 2),
        grid=(NB,),
        in_specs=specs,
        out_specs=out_specs,
        out_shape=out_shape,
        interpret=_INTERPRET,
    )(*ins)


_KH = 128
_NSUB = 16
_RPT = 624
_NTAIL = N - _NSUB * _RPT
_CHF = 80
_CHE = 125
_NBF = 4
_NBE = 2


def _splat(vec16, j):
    idx = jnp.full((16, 1), j, dtype=jnp.int32)
    return lax.gather(
        vec16, idx,
        dimension_numbers=lax.GatherDimensionNumbers(
            offset_dims=(), collapsed_slice_dims=(0,), start_index_map=(0,)),
        slice_sizes=(1,), mode=lax.GatherScatterMode.PROMISE_IN_BOUNDS)


def _spmm_body(mode, ch, nchunk, nb, sa_hbm, sb_hbm, src_hbm, dst_hbm, w_hbm,
               oa_hbm, ob_hbm, acc_sh, *scratch):
    bufs = scratch[0:nb]
    srcv = scratch[nb:2 * nb]
    dstv = scratch[2 * nb:3 * nb]
    wv = scratch[3 * nb:4 * nb]
    gsem = scratch[4 * nb:5 * nb]
    ssem = scratch[5 * nb:6 * nb]
    isem = scratch[6 * nb:7 * nb]
    c = lax.axis_index("c")
    s = lax.axis_index("s")
    row0 = pl.multiple_of(s * _RPT, 8)
    nrounds = nchunk // nb
    rem = nchunk - nrounds * nb
    wid = c * _NSUB + s if mode == "edge" else s

    def zrow(i, carry):
        for kk in range(8):
            bufs[0][i, pl.ds(kk * 16, 16)] = jnp.zeros((16,), jnp.float32)
        return carry
    lax.fori_loop(0, 16, zrow, 0)

    def zcopy(i, carry):
        off = pl.multiple_of(row0 + i * 16, 8)
        pltpu.sync_copy(bufs[0].at[pl.ds(0, 16)], acc_sh.at[pl.ds(off, 16)])
        return carry
    lax.fori_loop(0, _RPT // 16, zcopy, 0)

    @pl.when(s == 0)
    def _():
        pltpu.sync_copy(bufs[0].at[pl.ds(0, _NTAIL)],
                        acc_sh.at[pl.ds(_NSUB * _RPT, _NTAIL)])
    plsc.subcore_barrier()

    def issue_idx(b, ci, p):
        pltpu.async_copy(src_hbm.at[wid, ci], srcv[b].at[pl.ds(p, 1)],
                         isem[b])
        pltpu.async_copy(dst_hbm.at[wid, ci], dstv[b].at[pl.ds(p, 1)],
                         isem[b])
        pltpu.async_copy(w_hbm.at[wid, ci], wv[b].at[pl.ds(p, 1)],
                         isem[b])

    def wait_idx(b, ci, p):
        pltpu.make_async_copy(src_hbm.at[wid, ci],
                              srcv[b].at[pl.ds(p, 1)], isem[b]).wait()
        pltpu.make_async_copy(dst_hbm.at[wid, ci],
                              dstv[b].at[pl.ds(p, 1)], isem[b]).wait()
        pltpu.make_async_copy(w_hbm.at[wid, ci],
                              wv[b].at[pl.ds(p, 1)], isem[b]).wait()

    def issue_gather(b, p):
        idx = srcv[b].at[p, pl.ds(0, ch)]
        if mode == "edge":
            pltpu.async_copy(sa_hbm.at[idx], bufs[b], gsem[b])
        else:
            @pl.when(c == 0)
            def _():
                pltpu.async_copy(sa_hbm.at[idx], bufs[b], gsem[b])

            @pl.when(c == 1)
            def _():
                pltpu.async_copy(sb_hbm.at[idx], bufs[b], gsem[b])

    def wait_gather(b, p):
        pltpu.make_async_copy(sa_hbm.at[srcv[b].at[p, pl.ds(0, ch)]],
                              bufs[b], gsem[b]).wait()

    def issue_scatter(b, p):
        pltpu.async_copy(bufs[b], acc_sh.at[dstv[b].at[p, pl.ds(0, ch)]],
                         ssem[b], add=True)

    def wait_scatter(b, p):
        pltpu.make_async_copy(bufs[b], acc_sh.at[dstv[b].at[p, pl.ds(0, ch)]],
                              ssem[b]).wait()

    def scale(b, p):
        ngrp = ch // 16
        tail = ch % 16

        def grp(g, carry):
            w16 = wv[b][p, pl.ds(g * 16, 16)]
            e0 = g * 16
            for j in range(16):
                wj = _splat(w16, j)
                for kk in range(8):
                    bufs[b][e0 + j, pl.ds(kk * 16, 16)] = (
                        bufs[b][e0 + j, pl.ds(kk * 16, 16)] * wj)
            return carry
        lax.fori_loop(0, ngrp, grp, 0)
        if tail:
            w16 = wv[b][p, pl.ds(ngrp * 16, 16)]
            for j in range(tail):
                wj = _splat(w16, j)
                for kk in range(8):
                    bufs[b][ngrp * 16 + j, pl.ds(kk * 16, 16)] = (
                        bufs[b][ngrp * 16 + j, pl.ds(kk * 16, 16)] * wj)

    for b in range(nb):
        issue_idx(b, b, 0)
    for b in range(nb):
        issue_idx(b, nb + b, 1)
    for b in range(nb):
        wait_idx(b, b, 0)
        issue_gather(b, 0)

    def rnd(i0, carry):
        p = jnp.bitwise_and(i0, 1)
        p1 = jnp.bitwise_and(i0 + 1, 1)
        for b in range(nb):
            wait_gather(b, p)
            scale(b, p)
            issue_scatter(b, p)
        for b in range(nb):
            ci = i0 * nb + b
            wait_scatter(b, p)

            @pl.when(ci + 2 * nb < nchunk)
            def _(b=b, ci=ci, p=p):
                issue_idx(b, ci + 2 * nb, p)

            @pl.when(i0 < nrounds - 1)
            def _(b=b, ci=ci, p1=p1):
                wait_idx(b, ci + nb, p1)
                issue_gather(b, p1)
        return carry
    lax.fori_loop(0, nrounds, rnd, 0)

    for r in range(rem):
        ci = nrounds * nb + r
        p = nrounds & 1
        wait_idx(r, ci, p)
        issue_gather(r, p)
        wait_gather(r, p)
        scale(r, p)
        issue_scatter(r, p)
        wait_scatter(r, p)
    plsc.subcore_barrier()

    @pl.when(c == 0)
    def _():
        pltpu.sync_copy(acc_sh.at[pl.ds(row0, _RPT)],
                        oa_hbm.at[pl.ds(row0, _RPT)])

        @pl.when(s == 0)
        def _():
            pltpu.sync_copy(acc_sh.at[pl.ds(_NSUB * _RPT, _NTAIL)],
                            oa_hbm.at[pl.ds(_NSUB * _RPT, _NTAIL)])

    @pl.when(c == 1)
    def _():
        pltpu.sync_copy(acc_sh.at[pl.ds(row0, _RPT)],
                        ob_hbm.at[pl.ds(row0, _RPT)])

        @pl.when(s == 0)
        def _():
            pltpu.sync_copy(acc_sh.at[pl.ds(_NSUB * _RPT, _NTAIL)],
                            ob_hbm.at[pl.ds(_NSUB * _RPT, _NTAIL)])


def _spmm(mode, sa, sb, src3, dst3, w3):
    nworker = 32 if mode == "edge" else _NSUB
    ch = _CHE if mode == "edge" else _CHF
    nb = _NBE if mode == "edge" else _NBF
    nchunk = E // (nworker * ch)
    mesh = plsc.VectorSubcoreMesh(core_axis_name="c", subcore_axis_name="s")
    f = pl.kernel(
        functools.partial(_spmm_body, mode, ch, nchunk, nb),
        out_type=[jax.ShapeDtypeStruct((N, _KH), jnp.float32),
                  jax.ShapeDtypeStruct((N, _KH), jnp.float32)],
        mesh=mesh,
        scratch_types=[
            pltpu.VMEM_SHARED((N, _KH), jnp.float32),
        ] + [pltpu.VMEM((ch, _KH), jnp.float32) for _ in range(nb)]
          + [pltpu.VMEM((2, 128), jnp.int32) for _ in range(nb)]
          + [pltpu.VMEM((2, 128), jnp.int32) for _ in range(nb)]
          + [pltpu.VMEM((2, 128), jnp.float32) for _ in range(nb)]
          + [pltpu.SemaphoreType.DMA for _ in range(3 * nb)],
    )
    return f(sa, sb, src3, dst3, w3)


def _chunked(a, nw, ch):
    return jnp.pad(a.reshape(nw, -1, 1, ch),
                   ((0, 0), (0, 0), (0, 0), (0, 128 - ch)))


def kernel(x, edge_index, edge_weight,
           ae_enc1_W, ae_enc1_b, ae_enc2_W, ae_enc2_b, ae_enc3_W, ae_enc3_b,
           ae_z1_W, ae_z1_b, ae_z2_W, ae_z2_b,
           ae_dec0_W, ae_dec0_b, ae_dec1_W, ae_dec1_b, ae_dec2_W, ae_dec2_b,
           ae_dec3_W, ae_dec3_b, ae_xbar_W, ae_xbar_b,
           gcn1_W, gcn2_W, gcn3_W, gcn4_W,
           mlp_W, mlp_b):
    src_e = _chunked(edge_index[0], _NSUB, _CHF)
    dst_e = _chunked(edge_index[1], _NSUB, _CHF)
    w_e = _chunked(edge_weight, _NSUB, _CHF)
    src_w = _chunked(edge_index[0], 32, _CHE)
    dst_w = _chunked(edge_index[1], 32, _CHE)
    w_w = _chunked(edge_weight, 32, _CHE)

    x_bar = _ae(
        x,
        [ae_enc1_W, ae_enc2_W, ae_enc3_W, ae_z1_W, ae_z2_W,
         ae_dec0_W, ae_dec1_W, ae_dec2_W, ae_dec3_W, ae_xbar_W],
        [ae_enc1_b, ae_enc2_b, ae_enc3_b, ae_z1_b, ae_z2_b,
         ae_dec0_b, ae_dec1_b, ae_dec2_b, ae_dec3_b, ae_xbar_b],
    )

    g1a, g1b = _spmm("edge", x, x, src_w, dst_w, w_w)
    z1a, z1b = _dense("plain", "add", g1a, 0, g1b, 0, 128, None, gcn1_W)
    sums1 = _stats("cat", z1a, 0, z1b, 0, 256)

    s2a, s2b = _dense("norm", "cat", z1a, 0, z1b, 0, 256, sums1, gcn2_W)
    g2a, g2b = _spmm("feat", s2a, s2b, src_e, dst_e, w_e)
    sums2 = _stats("cat", g2a, 0, g2b, 0, 128)

    s3a, s3b = _dense("norm", "cat", g2a, 0, g2b, 0, 128, sums2, gcn3_W)
    g3a, g3b = _spmm("feat", s3a, s3b, src_e, dst_e, w_e)
    sums3 = _stats("cat", g3a, 0, g3b, 0, 128)

    W4p = jnp.pad(gcn4_W, ((0, 0), (0, _KH - gcn4_W.shape[1])))
    s4 = _dense("single", "cat", g3a, 0, g3b, 0, 128, sums3, W4p)
    g4a, g4b = _spmm("edge", s4, s4, src_w, dst_w, w_w)
    sums4 = _stats("add", g4a, 0, g4b, 0, 128)

    mlp_Wp = jnp.pad(mlp_W, ((0, _KH - mlp_W.shape[0]), (0, 0)))
    predict = _dense("head", "add", g4a, 0, g4b, 0, 128, sums4, mlp_Wp, mlp_b)
    return jnp.concatenate([x_bar, predict], axis=1)
